# baseline (speedup 1.0000x reference)
import functools

import jax
import jax.numpy as jnp
from jax import lax
from jax.experimental import pallas as pl
from jax.experimental.pallas import tpu as pltpu
from jax.experimental.pallas import tpu_sc as plsc

_B, _S = 16384, 50
_D = 32
_N_IDX = _B * _S
_CHUNK = 128
_N_ROWS = _N_IDX // _CHUNK

_info = plsc.get_sparse_core_info()
_NC, _NS = _info.num_cores, _info.num_subcores
_NW = _NC * _NS

_ROWS_PER_W = _N_ROWS // _NW
_K = 10
_STEP_ROWS = _K * _CHUNK
_N_STEPS = _ROWS_PER_W // _K
_N_OUTER = _N_STEPS // 2


def _emb_body(table, idx, out, idx_v, rows_v, sem0, sem1):
    wid = lax.axis_index("s") * _NC + lax.axis_index("c")
    base_row = wid * _ROWS_PER_W
    sems = (sem0, sem1)

    def load_fire(s, b):
        pltpu.sync_copy(idx.at[pl.ds(base_row + s * _K, _K)], idx_v.at[b])
        for j in range(_K):
            pltpu.async_copy(
                table.at[idx_v.at[b, j]],
                rows_v.at[b, pl.ds(j * _CHUNK, _CHUNK)],
                sems[b],
            )

    def drain_store(s, b):
        pltpu.make_async_copy(
            table.at[pl.ds(0, _STEP_ROWS)], rows_v.at[b], sems[b]
        ).wait()
        pltpu.sync_copy(
            rows_v.at[b],
            out.at[pl.ds((base_row + s * _K) * _CHUNK, _STEP_ROWS)],
        )

    load_fire(0, 0)

    def outer(t, carry):
        s0 = 2 * t
        load_fire(s0 + 1, 1)
        drain_store(s0, 0)

        @pl.when(t < _N_OUTER - 1)
        def _():
            load_fire(s0 + 2, 0)

        drain_store(s0 + 1, 1)
        return carry

    lax.fori_loop(0, _N_OUTER, outer, 0)


@functools.partial(
    pl.kernel,
    mesh=plsc.VectorSubcoreMesh(core_axis_name="c", subcore_axis_name="s"),
    out_type=jax.ShapeDtypeStruct((_N_IDX, _D), jnp.float32),
    scratch_types=[
        pltpu.VMEM((2, _K, _CHUNK), jnp.int32),
        pltpu.VMEM((2, _STEP_ROWS, _D), jnp.float32),
        pltpu.SemaphoreType.DMA,
        pltpu.SemaphoreType.DMA,
    ],
    compiler_params=pltpu.CompilerParams(use_tc_tiling_on_sc=False),
)
def _emb_kernel(table, idx, out, idx_v, rows_v, sem0, sem1):
    _emb_body(table, idx, out, idx_v, rows_v, sem0, sem1)


def kernel(token_ids, weight):
    idx = token_ids.astype(jnp.int32).reshape(_N_ROWS, _CHUNK)
    out = _emb_kernel(weight, idx)
    return out.reshape(_B, _S, _D)


# speedup vs baseline: 1.1591x; 1.1591x over previous
import functools

import jax
import jax.numpy as jnp
from jax import lax
from jax.experimental import pallas as pl
from jax.experimental.pallas import tpu as pltpu
from jax.experimental.pallas import tpu_sc as plsc

_B, _S = 16384, 50
_D = 32
_N_IDX = _B * _S
_CHUNK = 128
_N_ROWS = _N_IDX // _CHUNK

_info = plsc.get_sparse_core_info()
_NC, _NS = _info.num_cores, _info.num_subcores
_NW = _NC * _NS

_ROWS_PER_W = _N_ROWS // _NW
_K = 8
_N_STEPS = _ROWS_PER_W // _K


def _emb_body(table, idx, out, idx_v, rows_v, sem):
    wid = lax.axis_index("s") * _NC + lax.axis_index("c")
    base = wid * _ROWS_PER_W

    def step(g, carry):
        row_off = base + g * _K
        pltpu.sync_copy(idx.at[pl.ds(row_off, _K)], idx_v)
        copies = [
            pltpu.async_copy(table.at[idx_v.at[j]], rows_v.at[j], sem)
            for j in range(_K)
        ]
        for c in copies:
            c.wait()
        pltpu.sync_copy(rows_v, out.at[pl.ds(row_off, _K)])
        return carry

    lax.fori_loop(0, _N_STEPS, step, 0)


@functools.partial(
    pl.kernel,
    mesh=plsc.VectorSubcoreMesh(core_axis_name="c", subcore_axis_name="s"),
    out_type=jax.ShapeDtypeStruct((_N_ROWS, _CHUNK, _D), jnp.float32),
    scratch_types=[
        pltpu.VMEM((_K, _CHUNK), jnp.int32),
        pltpu.VMEM((_K, _CHUNK, _D), jnp.float32),
        pltpu.SemaphoreType.DMA,
    ],
    compiler_params=pltpu.CompilerParams(use_tc_tiling_on_sc=False),
)
def _emb_kernel(table, idx, out, idx_v, rows_v, sem):
    _emb_body(table, idx, out, idx_v, rows_v, sem)


def kernel(token_ids, weight):
    idx = token_ids.astype(jnp.int32).reshape(_N_ROWS, _CHUNK)
    out = _emb_kernel(weight, idx)
    return out.reshape(_B, _S, _D)


# speedup vs baseline: 1.5714x; 1.3557x over previous
import functools

import jax
import jax.numpy as jnp
from jax import lax
from jax.experimental import pallas as pl
from jax.experimental.pallas import tpu as pltpu
from jax.experimental.pallas import tpu_sc as plsc

_B, _S = 16384, 50
_D = 32

_info = plsc.get_sparse_core_info()
_NC, _NS = _info.num_cores, _info.num_subcores
_NW = _NC * _NS

_ROWS_PER_W = _B // _NW
_K = 16
_N_STEPS = _ROWS_PER_W // _K


def _emb_body(idx, table, out, idx_v, rows_v, sem):
    wid = lax.axis_index("s") * _NC + lax.axis_index("c")
    base = wid * _ROWS_PER_W

    def step(g, carry):
        row_off = base + g * _K
        pltpu.sync_copy(idx.at[pl.ds(row_off, _K)], idx_v)
        copies = [
            pltpu.async_copy(table.at[idx_v.at[j]], rows_v.at[j], sem)
            for j in range(_K)
        ]
        for c in copies:
            c.wait()
        pltpu.sync_copy(rows_v, out.at[pl.ds(row_off, _K)])
        return carry

    lax.fori_loop(0, _N_STEPS, step, 0)


@functools.partial(
    pl.kernel,
    mesh=plsc.VectorSubcoreMesh(core_axis_name="c", subcore_axis_name="s"),
    out_type=jax.ShapeDtypeStruct((_B, _S, _D), jnp.float32),
    scratch_types=[
        pltpu.VMEM((_K, _S), jnp.int32),
        pltpu.VMEM((_K, _S, _D), jnp.float32),
        pltpu.SemaphoreType.DMA,
    ],
    compiler_params=pltpu.CompilerParams(use_tc_tiling_on_sc=False),
)
def _emb_kernel(idx, table, out, idx_v, rows_v, sem):
    _emb_body(idx, table, out, idx_v, rows_v, sem)


def kernel(token_ids, weight):
    return _emb_kernel(token_ids.astype(jnp.int32), weight)


# speedup vs baseline: 2.1776x; 1.3858x over previous
import functools

import jax
import jax.numpy as jnp
from jax import lax
from jax.experimental import pallas as pl
from jax.experimental.pallas import tpu as pltpu
from jax.experimental.pallas import tpu_sc as plsc

_B, _S = 16384, 50
_D = 32
_SP, _DP = 56, 128

_info = plsc.get_sparse_core_info()
_NC, _NS = _info.num_cores, _info.num_subcores
_NW = _NC * _NS

_ROWS_PER_W = _B // _NW
_K = 16
_N_STEPS = _ROWS_PER_W // _K


def _emb_body(idx, table, out, idx_v, rows_v, sem):
    wid = lax.axis_index("s") * _NC + lax.axis_index("c")
    base = wid * _ROWS_PER_W

    def step(g, carry):
        row_off = base + g * _K
        pltpu.sync_copy(idx.at[pl.ds(row_off, _K)], idx_v)
        copies = [
            pltpu.async_copy(table.at[idx_v.at[j]], rows_v.at[j], sem)
            for j in range(_K)
        ]
        for c in copies:
            c.wait()
        pltpu.sync_copy(
            rows_v,
            out.at[pl.ds(row_off, _K), pl.ds(0, _S), pl.ds(0, _D)],
        )
        return carry

    lax.fori_loop(0, _N_STEPS, step, 0)


@functools.partial(
    pl.kernel,
    mesh=plsc.VectorSubcoreMesh(core_axis_name="c", subcore_axis_name="s"),
    out_type=jax.ShapeDtypeStruct((_B, _SP, _DP), jnp.float32),
    scratch_types=[
        pltpu.VMEM((_K, _S), jnp.int32),
        pltpu.VMEM((_K, _S, _D), jnp.float32),
        pltpu.SemaphoreType.DMA,
    ],
    compiler_params=pltpu.CompilerParams(use_tc_tiling_on_sc=False),
)
def _emb_kernel(idx, table, out, idx_v, rows_v, sem):
    _emb_body(idx, table, out, idx_v, rows_v, sem)


def kernel(token_ids, weight):
    out = _emb_kernel(token_ids.astype(jnp.int32), weight)
    return out[:, :_S, :_D]
